# baseline (speedup 1.0000x reference)
import functools

import jax
import jax.numpy as jnp
from jax import lax
from jax.experimental import pallas as pl
from jax.experimental.pallas import tpu as pltpu
from jax.experimental.pallas import tpu_sc as plsc

MAX_N = 119
DIM = 64
NC, NS = 2, 16
NW = NC * NS

NB = 4096
NI = 200
BPW = NB // NW
GRP = BPW // 16
L = 16
TI = NI // 8


def _emb_body(xt_hbm, tt_hbm, out_hbm, x_tile, t_tile, ob0, ob1, sem0, sem1):
    c = lax.axis_index("c")
    s = lax.axis_index("s")
    wid = s * NC + c

    pltpu.sync_copy(tt_hbm, t_tile)
    pltpu.sync_copy(xt_hbm.at[:, wid], x_tile)

    def build_block(i, ob):
        ti = i // 8
        si = i % 8
        idxm1 = [
            x_tile[ti, si, pl.ds(g * L, L)] - jnp.full((L,), 1, jnp.int32)
            for g in range(GRP)
        ]
        for j in range(DIM):
            jv = jnp.full((L,), j, jnp.int32)
            for g in range(GRP):
                ob[j // 8, j % 8, pl.ds(g * L, L)] = plsc.load_gather(
                    t_tile, [jv, idxm1[g]]
                )

    def pair(p, carry):
        i0 = 2 * p

        @pl.when(p > 0)
        def _():
            pltpu.make_async_copy(ob0, out_hbm.at[i0, :, wid], sem0).wait()

        build_block(i0, ob0)
        pltpu.async_copy(ob0, out_hbm.at[i0, :, wid], sem0)

        @pl.when(p > 0)
        def _():
            pltpu.make_async_copy(ob1, out_hbm.at[i0 + 1, :, wid], sem1).wait()

        build_block(i0 + 1, ob1)
        pltpu.async_copy(ob1, out_hbm.at[i0 + 1, :, wid], sem1)
        return carry

    lax.fori_loop(0, NI // 2, pair, 0)
    pltpu.make_async_copy(ob0, out_hbm.at[NI - 2, :, wid], sem0).wait()
    pltpu.make_async_copy(ob1, out_hbm.at[NI - 1, :, wid], sem1).wait()


@jax.jit
def _emb_call(xt, tt):
    mesh = plsc.VectorSubcoreMesh(core_axis_name="c", subcore_axis_name="s")
    run = pl.kernel(
        _emb_body,
        out_type=jax.ShapeDtypeStruct((NI, 8, NW, 8, BPW), jnp.float32),
        mesh=mesh,
        compiler_params=pltpu.CompilerParams(
            use_tc_tiling_on_sc=False, needs_layout_passes=False),
        scratch_types=[
            pltpu.VMEM((TI, 8, BPW), jnp.int32),
            pltpu.VMEM((DIM, MAX_N), jnp.float32),
            pltpu.VMEM((8, 8, BPW), jnp.float32),
            pltpu.VMEM((8, 8, BPW), jnp.float32),
            pltpu.SemaphoreType.DMA,
            pltpu.SemaphoreType.DMA,
        ],
    )
    return run(xt, tt)


def kernel(x, table):
    xt = (x.astype(jnp.int32)
          .reshape(NW, BPW, TI, 8)
          .transpose(2, 0, 3, 1))
    tt = table.T
    out5 = _emb_call(xt, tt)
    return (out5.transpose(0, 1, 3, 2, 4)
            .reshape(NI, DIM, NB)
            .transpose(2, 0, 1))


# speedup vs baseline: 1.0011x; 1.0011x over previous
import functools

import jax
import jax.numpy as jnp
from jax import lax
from jax.experimental import pallas as pl
from jax.experimental.pallas import tpu as pltpu
from jax.experimental.pallas import tpu_sc as plsc

MAX_N = 119
DIM = 64
NC, NS = 2, 16
NW = NC * NS

NB = 4096
NI = 200
BPW = NB // NW
GRP = BPW // 16
L = 16
TI = NI // 8


def _emb_body(xt_hbm, tt_hbm, out_hbm, x_tile, t_tile, ob0, ob1, sem0, sem1):
    c = lax.axis_index("c")
    s = lax.axis_index("s")
    wid = s * NC + c

    pltpu.sync_copy(tt_hbm, t_tile)
    pltpu.sync_copy(xt_hbm.at[:, wid], x_tile)

    def build_block(i, ob):
        ti = i // 8
        si = i % 8
        addr = [
            x_tile[ti, si, pl.ds(g * L, L)] - jnp.full((L,), 1, jnp.int32)
            for g in range(GRP)
        ]
        step = jnp.full((L,), MAX_N, jnp.int32)
        for j in range(DIM):
            for g in range(GRP):
                ob[j // 8, j % 8, pl.ds(g * L, L)] = plsc.load_gather(
                    t_tile, [addr[g]]
                )
            if j < DIM - 1:
                addr = [a + step for a in addr]

    def pair(p, carry):
        i0 = 2 * p

        @pl.when(p > 0)
        def _():
            pltpu.make_async_copy(ob0, out_hbm.at[i0, :, wid], sem0).wait()

        build_block(i0, ob0)
        pltpu.async_copy(ob0, out_hbm.at[i0, :, wid], sem0)

        @pl.when(p > 0)
        def _():
            pltpu.make_async_copy(ob1, out_hbm.at[i0 + 1, :, wid], sem1).wait()

        build_block(i0 + 1, ob1)
        pltpu.async_copy(ob1, out_hbm.at[i0 + 1, :, wid], sem1)
        return carry

    lax.fori_loop(0, NI // 2, pair, 0)
    pltpu.make_async_copy(ob0, out_hbm.at[NI - 2, :, wid], sem0).wait()
    pltpu.make_async_copy(ob1, out_hbm.at[NI - 1, :, wid], sem1).wait()


@jax.jit
def _emb_call(xt, tt):
    mesh = plsc.VectorSubcoreMesh(core_axis_name="c", subcore_axis_name="s")
    run = pl.kernel(
        _emb_body,
        out_type=jax.ShapeDtypeStruct((NI, 8, NW, 8, BPW), jnp.float32),
        mesh=mesh,
        compiler_params=pltpu.CompilerParams(
            use_tc_tiling_on_sc=False, needs_layout_passes=False),
        scratch_types=[
            pltpu.VMEM((TI, 8, BPW), jnp.int32),
            pltpu.VMEM((DIM * MAX_N,), jnp.float32),
            pltpu.VMEM((8, 8, BPW), jnp.float32),
            pltpu.VMEM((8, 8, BPW), jnp.float32),
            pltpu.SemaphoreType.DMA,
            pltpu.SemaphoreType.DMA,
        ],
    )
    return run(xt, tt)


def kernel(x, table):
    xt = (x.astype(jnp.int32)
          .reshape(NW, BPW, TI, 8)
          .transpose(2, 0, 3, 1))
    tt = table.T.reshape(-1)
    out5 = _emb_call(xt, tt)
    return (out5.transpose(0, 1, 3, 2, 4)
            .reshape(NI, DIM, NB)
            .transpose(2, 0, 1))
